# baseline (speedup 1.0000x reference)
import functools

import jax
import jax.numpy as jnp
from jax import lax
from jax.experimental import pallas as pl
from jax.experimental.pallas import tpu as pltpu
from jax.experimental.pallas import tpu_sc as plsc

FEAT = 128
BATCH = 16384
MOM = 0.3
HALF = BATCH // 2

_info = plsc.get_sparse_core_info()
_NC, _NS = _info.num_cores, _info.num_subcores
_NW = _NC * _NS
_B_PER_W = HALF // _NW
_CHUNK = 128


def _sc_gather_body(half_off, idx_hbm, mu_hbm, tau_hbm, mu_out, tau_out,
                    idx0, idx1, mu_a, tau_a, mu_b, tau_b,
                    sem_ga, sem_gb, sem_s):
    wid = lax.axis_index("s") * _NC + lax.axis_index("c")
    base = wid * _B_PER_W
    src = half_off + base
    pltpu.sync_copy(idx_hbm.at[pl.ds(src, _CHUNK)], idx0)
    g0m = pltpu.async_copy(mu_hbm.at[idx0], mu_a, sem_ga)
    g0t = pltpu.async_copy(tau_hbm.at[idx0], tau_a, sem_ga)
    pltpu.sync_copy(idx_hbm.at[pl.ds(src + _CHUNK, _CHUNK)], idx1)
    g1m = pltpu.async_copy(mu_hbm.at[idx1], mu_b, sem_gb)
    g1t = pltpu.async_copy(tau_hbm.at[idx1], tau_b, sem_gb)
    g0m.wait(); g0t.wait()
    s0m = pltpu.async_copy(mu_a, mu_out.at[pl.ds(base, _CHUNK)], sem_s)
    s0t = pltpu.async_copy(tau_a, tau_out.at[pl.ds(base, _CHUNK)], sem_s)
    g1m.wait(); g1t.wait()
    s1m = pltpu.async_copy(mu_b, mu_out.at[pl.ds(base + _CHUNK, _CHUNK)], sem_s)
    s1t = pltpu.async_copy(tau_b, tau_out.at[pl.ds(base + _CHUNK, _CHUNK)], sem_s)
    s0m.wait(); s0t.wait(); s1m.wait(); s1t.wait()


def _make_sc_gather(half_off):
    return functools.partial(
        pl.kernel,
        mesh=plsc.VectorSubcoreMesh(core_axis_name="c", subcore_axis_name="s"),
        out_type=[
            jax.ShapeDtypeStruct((HALF, FEAT), jnp.float32),
            jax.ShapeDtypeStruct((HALF, FEAT), jnp.float32),
        ],
        scratch_types=[
            pltpu.VMEM((_CHUNK,), jnp.int32),
            pltpu.VMEM((_CHUNK,), jnp.int32),
            pltpu.VMEM((_CHUNK, FEAT), jnp.float32),
            pltpu.VMEM((_CHUNK, FEAT), jnp.float32),
            pltpu.VMEM((_CHUNK, FEAT), jnp.float32),
            pltpu.VMEM((_CHUNK, FEAT), jnp.float32),
            pltpu.SemaphoreType.DMA,
            pltpu.SemaphoreType.DMA,
            pltpu.SemaphoreType.DMA,
        ],
    )(functools.partial(_sc_gather_body, half_off))


_sc_gather_h0 = _make_sc_gather(0)
_sc_gather_h1 = _make_sc_gather(HALF)

_BLK = 512
_HBLKS = HALF // _BLK


def _tc_norm0_body(z_ref, mu_ref, tau_ref, o_ref):
    z = z_ref[...]
    sigma = jnp.log1p(jnp.exp(tau_ref[...]))
    z_hat = (z - mu_ref[...]) / sigma
    o_ref[...] = (1.0 - MOM) * z_hat + MOM * z


def _tc_norm1_body(z_ref, mu_ref, tau_ref, prev_ref, o_ref):
    _tc_norm0_body(z_ref, mu_ref, tau_ref, o_ref)


def _tc_norm0(z, mu_a, tau_a):
    half_spec = pl.BlockSpec((_BLK, FEAT), lambda i: (i, 0))
    return pl.pallas_call(
        _tc_norm0_body,
        grid=(_HBLKS,),
        in_specs=[half_spec, half_spec, half_spec],
        out_specs=half_spec,
        out_shape=jax.ShapeDtypeStruct((BATCH, FEAT), jnp.float32),
    )(z, mu_a, tau_a)


def _tc_norm1(z, mu_a, tau_a, prev):
    half_spec = pl.BlockSpec((_BLK, FEAT), lambda i: (i, 0))
    off_spec = pl.BlockSpec((_BLK, FEAT), lambda i: (i + _HBLKS, 0))
    any_spec = pl.BlockSpec(memory_space=pl.ANY)
    return pl.pallas_call(
        _tc_norm1_body,
        grid=(_HBLKS,),
        in_specs=[off_spec, half_spec, half_spec, any_spec],
        out_specs=off_spec,
        out_shape=jax.ShapeDtypeStruct((BATCH, FEAT), jnp.float32),
        input_output_aliases={3: 0},
    )(z, mu_a, tau_a, prev)


def kernel(z, a, mu, tau):
    a32 = a.astype(jnp.int32)
    mu_a0, tau_a0 = _sc_gather_h0(a32, mu, tau)
    mu_a1, tau_a1 = _sc_gather_h1(a32, mu, tau)
    out = _tc_norm0(z, mu_a0, tau_a0)
    out = _tc_norm1(z, mu_a1, tau_a1, out)
    return out


# speedup vs baseline: 1.2239x; 1.2239x over previous
import functools

import jax
import jax.numpy as jnp
from jax import lax
from jax.experimental import pallas as pl
from jax.experimental.pallas import tpu as pltpu
from jax.experimental.pallas import tpu_sc as plsc

FEAT = 128
BATCH = 16384
MOM = 0.3
HALF = BATCH // 2

_info = plsc.get_sparse_core_info()
_NC, _NS = _info.num_cores, _info.num_subcores
_NW = _NC * _NS
_B_PER_W = HALF // _NW
_CHUNK = 128


def _sc_gather_body(half_off, idx_hbm, mu_hbm, tau_hbm, mu_out, tau_out,
                    idx0, idx1, mu_a, tau_a, mu_b, tau_b,
                    sem_ga, sem_gb, sem_s):
    wid = lax.axis_index("s") * _NC + lax.axis_index("c")
    base = wid * _B_PER_W
    src = half_off + base
    pltpu.sync_copy(idx_hbm.at[pl.ds(src, _CHUNK)], idx0)
    g0m = pltpu.async_copy(mu_hbm.at[idx0], mu_a, sem_ga)
    g0t = pltpu.async_copy(tau_hbm.at[idx0], tau_a, sem_ga)
    pltpu.sync_copy(idx_hbm.at[pl.ds(src + _CHUNK, _CHUNK)], idx1)
    g1m = pltpu.async_copy(mu_hbm.at[idx1], mu_b, sem_gb)
    g1t = pltpu.async_copy(tau_hbm.at[idx1], tau_b, sem_gb)
    g0m.wait(); g0t.wait()
    s0m = pltpu.async_copy(mu_a, mu_out.at[pl.ds(base, _CHUNK)], sem_s)
    s0t = pltpu.async_copy(tau_a, tau_out.at[pl.ds(base, _CHUNK)], sem_s)
    g1m.wait(); g1t.wait()
    s1m = pltpu.async_copy(mu_b, mu_out.at[pl.ds(base + _CHUNK, _CHUNK)], sem_s)
    s1t = pltpu.async_copy(tau_b, tau_out.at[pl.ds(base + _CHUNK, _CHUNK)], sem_s)
    s0m.wait(); s0t.wait(); s1m.wait(); s1t.wait()


def _make_sc_gather(half_off):
    return functools.partial(
        pl.kernel,
        mesh=plsc.VectorSubcoreMesh(core_axis_name="c", subcore_axis_name="s"),
        out_type=[
            jax.ShapeDtypeStruct((HALF, FEAT), jnp.float32),
            jax.ShapeDtypeStruct((HALF, FEAT), jnp.float32),
        ],
        scratch_types=[
            pltpu.VMEM((_CHUNK,), jnp.int32),
            pltpu.VMEM((_CHUNK,), jnp.int32),
            pltpu.VMEM((_CHUNK, FEAT), jnp.float32),
            pltpu.VMEM((_CHUNK, FEAT), jnp.float32),
            pltpu.VMEM((_CHUNK, FEAT), jnp.float32),
            pltpu.VMEM((_CHUNK, FEAT), jnp.float32),
            pltpu.SemaphoreType.DMA,
            pltpu.SemaphoreType.DMA,
            pltpu.SemaphoreType.DMA,
        ],
    )(functools.partial(_sc_gather_body, half_off))


_sc_gather_h0 = _make_sc_gather(0)
_sc_gather_h1 = _make_sc_gather(HALF)

_BLK = 4096
_HBLKS = HALF // _BLK


def _tc_norm0_body(z_ref, mu_ref, tau_ref, o_ref):
    z = z_ref[...]
    sigma = jnp.log1p(jnp.exp(tau_ref[...]))
    z_hat = (z - mu_ref[...]) / sigma
    o_ref[...] = (1.0 - MOM) * z_hat + MOM * z


def _tc_norm1_body(z_ref, mu_ref, tau_ref, prev_ref, o_ref):
    _tc_norm0_body(z_ref, mu_ref, tau_ref, o_ref)


def _tc_norm0(z, mu_a, tau_a):
    half_spec = pl.BlockSpec((_BLK, FEAT), lambda i: (i, 0))
    return pl.pallas_call(
        _tc_norm0_body,
        grid=(_HBLKS,),
        in_specs=[half_spec, half_spec, half_spec],
        out_specs=half_spec,
        out_shape=jax.ShapeDtypeStruct((BATCH, FEAT), jnp.float32),
    )(z, mu_a, tau_a)


def _tc_norm1(z, mu_a, tau_a, prev):
    half_spec = pl.BlockSpec((_BLK, FEAT), lambda i: (i, 0))
    off_spec = pl.BlockSpec((_BLK, FEAT), lambda i: (i + _HBLKS, 0))
    any_spec = pl.BlockSpec(memory_space=pl.ANY)
    return pl.pallas_call(
        _tc_norm1_body,
        grid=(_HBLKS,),
        in_specs=[off_spec, half_spec, half_spec, any_spec],
        out_specs=off_spec,
        out_shape=jax.ShapeDtypeStruct((BATCH, FEAT), jnp.float32),
        input_output_aliases={3: 0},
    )(z, mu_a, tau_a, prev)


def kernel(z, a, mu, tau):
    a32 = a.astype(jnp.int32)
    mu_a0, tau_a0 = _sc_gather_h0(a32, mu, tau)
    mu_a1, tau_a1 = _sc_gather_h1(a32, mu, tau)
    out = _tc_norm0(z, mu_a0, tau_a0)
    out = _tc_norm1(z, mu_a1, tau_a1, out)
    return out
